# baseline (speedup 1.0000x reference)
import functools

import jax
import jax.numpy as jnp
from jax import lax
from jax.experimental import pallas as pl
from jax.experimental.pallas import tpu as pltpu
from jax.experimental.pallas import tpu_sc as plsc

DIM = 64
P = 20
NEG = 50
REAL = P + NEG
IDXW = 72
DOTW = 80
NC = 2
NS = 16
NW = NC * NS
L = 16
NBUF = 2
CB = 64


@functools.lru_cache(maxsize=None)
def _sc_dots_fn(B: int, VP: int):
    CH = B // NW

    mesh = plsc.VectorSubcoreMesh(core_axis_name="c", subcore_axis_name="s")

    @functools.partial(
        pl.kernel,
        out_type=jax.ShapeDtypeStruct((B * DOTW,), jnp.float32),
        mesh=mesh,
        scratch_types=[
            pltpu.VMEM((CH,), jnp.int32),
            pltpu.VMEM((CH,), jnp.int32),
            pltpu.VMEM((CH * IDXW,), jnp.int32),
            pltpu.VMEM((CH * L,), jnp.int32),
            pltpu.VMEM((CH * DIM,), jnp.float32),
            pltpu.VMEM((CB, 2 * DIM), jnp.float32),
            pltpu.VMEM((NBUF, IDXW, 2 * DIM), jnp.float32),
            pltpu.VMEM((NBUF * DOTW,), jnp.float32),
            pltpu.VMEM((L * L,), jnp.float32),
            *([pltpu.SemaphoreType.DMA] * (2 * NBUF)),
        ],
        compiler_params=pltpu.CompilerParams(
            needs_layout_passes=False, use_tc_tiling_on_sc=True),
    )
    def sc_dots(cenj_hbm, cenp_hbm, ctxj_hbm, pbits_hbm, t_in, t_out,
                dots_hbm, cidx_v, cpar_v, ctxj_v, pb_v, cmat_v, ctemp_v,
                rows_v, dots_v, t_v, *sems):
        gsem = sems[:NBUF]
        osem = sems[NBUF:]
        wid = lax.axis_index("s") * NC + lax.axis_index("c")
        base = wid * CH
        lane = lax.iota(jnp.int32, L)

        pltpu.sync_copy(cenj_hbm.at[pl.ds(base, CH)], cidx_v)
        pltpu.sync_copy(cenp_hbm.at[pl.ds(base, CH)], cpar_v)
        pltpu.sync_copy(ctxj_hbm.at[pl.ds(base * IDXW, CH * IDXW)], ctxj_v)
        pltpu.sync_copy(pbits_hbm.at[pl.ds(base * L, CH * L)], pb_v)

        def cen_body(k, carry):
            q0 = k * CB
            pltpu.sync_copy(t_in.at[cidx_v.at[pl.ds(q0, CB)]], ctemp_v)
            for s in range(CB // L):
                parv = cpar_v[pl.ds(q0 + L * s, L)]
                for qq in range(L):
                    q = L * s + qq
                    sel = parv[qq] == 1
                    for t in range(DIM // L):
                        lo = ctemp_v[q, pl.ds(L * t, L)]
                        hi = ctemp_v[q, pl.ds(DIM + L * t, L)]
                        cmat_v[pl.ds((q0 + q) * DIM + L * t, L)] = jnp.where(
                            sel, hi, lo)
            return carry

        lax.fori_loop(0, CH // CB, cen_body, 0)

        def start(j, b):
            pltpu.async_copy(t_out.at[ctxj_v.at[pl.ds(b * IDXW, IDXW)]],
                             rows_v.at[j], gsem[j])

        def wait(j, b):
            pltpu.make_async_copy(t_out.at[ctxj_v.at[pl.ds(b * IDXW, IDXW)]],
                                  rows_v.at[j], gsem[j]).wait()

        def start_out(j, b):
            pltpu.async_copy(dots_v.at[pl.ds(j * DOTW, DOTW)],
                             dots_hbm.at[pl.ds((base + b) * DOTW, DOTW)],
                             osem[j])

        def wait_out(j, b):
            pltpu.make_async_copy(dots_v.at[pl.ds(j * DOTW, DOTW)],
                                  dots_hbm.at[pl.ds((base + b) * DOTW, DOTW)],
                                  osem[j]).wait()

        last = lane * L + (L - 1)

        def compute(j, b):
            c = [cmat_v[pl.ds(b * DIM + L * t, L)] for t in range(DIM // L)]
            pwv = pb_v[pl.ds(b * L, L)]
            pw = [pwv[w] for w in range((REAL + 31) // 32)]
            for g in range(DOTW // L):
                for q in range(min(REAL - g * L, L)):
                    r = g * L + q
                    bit = (pw[r // 32] >> (r % 32)) & 1
                    sel = bit == 1
                    p = None
                    for t in range(DIM // L):
                        lo = rows_v[j, r, pl.ds(L * t, L)]
                        hi = rows_v[j, r, pl.ds(DIM + L * t, L)]
                        term = jnp.where(sel, hi, lo) * c[t]
                        p = term if p is None else p + term
                    t_v[pl.ds(L * q, L)] = plsc.cumsum(p)
                dots_v[pl.ds(j * DOTW + g * L, L)] = plsc.load_gather(
                    t_v, [last])

        for j in range(NBUF):
            start(j, j)

        def body(i, carry):
            b0 = i * NBUF
            for j in range(NBUF):
                b = b0 + j
                wait(j, b)

                @pl.when(b >= NBUF)
                def _():
                    wait_out(j, b - NBUF)

                compute(j, b)
                start_out(j, b)

                @pl.when(b + NBUF < CH)
                def _():
                    start(j, b + NBUF)
            return carry

        lax.fori_loop(0, CH // NBUF, body, 0)
        for j in range(NBUF):
            wait_out(j, CH - NBUF + j)

    return sc_dots


def _tc_loss_body(d_ref, o_ref):
    x = d_ref[...]
    col = lax.broadcasted_iota(jnp.int32, x.shape, 1)
    y = x * jnp.where(col < P, 1.0, -1.0).astype(jnp.float32)
    ls = jnp.minimum(y, 0.0) - jnp.log1p(jnp.exp(-jnp.abs(y)))
    contrib = jnp.where(col < REAL, ls, 0.0)
    o_ref[...] = -jnp.sum(contrib, axis=1, keepdims=True)


@functools.lru_cache(maxsize=None)
def _tc_loss_fn(B: int):
    BT = 2048
    return pl.pallas_call(
        _tc_loss_body,
        grid=(B // BT,),
        in_specs=[pl.BlockSpec((BT, DOTW), lambda i: (i, 0))],
        out_specs=pl.BlockSpec((BT, 1), lambda i: (i, 0)),
        out_shape=jax.ShapeDtypeStruct((B, 1), jnp.float32),
    )


def kernel(center, positive, negative, in_emb, out_emb):
    B = center.shape[0]
    vocab = out_emb.shape[0]
    pad = (jnp.arange(B, dtype=jnp.int32)[:, None] * (IDXW - REAL)
           + jnp.arange(IDXW - REAL, dtype=jnp.int32)[None, :]) % vocab
    ctx = jnp.concatenate(
        [positive.astype(jnp.int32), negative.astype(jnp.int32), pad], axis=1)
    center = center.astype(jnp.int32)

    t_in = jnp.reshape(in_emb, (vocab // 2, 2 * DIM))
    t_out = jnp.reshape(out_emb, (vocab // 2, 2 * DIM))
    ctx_j = ctx >> 1
    par = ctx & 1
    shift = (jnp.arange(IDXW, dtype=jnp.int32) % 32)[None, :]
    word = jnp.arange(IDXW, dtype=jnp.int32) // 32
    bits = par << shift
    pbits = jnp.stack(
        [jnp.sum(jnp.where(word[None, :] == w, bits, 0), axis=1)
         for w in range((IDXW + 31) // 32)], axis=1)
    pbits = jnp.pad(pbits, ((0, 0), (0, 16 - pbits.shape[1])))

    dots = _sc_dots_fn(B, vocab // 2)(
        center >> 1, center & 1, ctx_j.reshape(-1), pbits.reshape(-1),
        t_in, t_out)
    return _tc_loss_fn(B)(dots.reshape(B, DOTW))
